# Initial kernel scaffold; baseline (speedup 1.0000x reference)
import jax
import jax.numpy as jnp
from jax.experimental import pallas as pl


def kernel(x, p_grid):
    raise NotImplementedError("write your pallas kernel here")


# speedup vs baseline: 1.7691x; 1.7691x over previous
import functools

import jax
import jax.numpy as jnp
from jax.experimental import pallas as pl
from jax.experimental.pallas import tpu as pltpu

_RADIUS2 = 0.0625
_K_NB = 10
_K = 8192
_Q_BLK = 400


def _bq_body(q_ref, pts_ref, ptsT_ref, map_ref, outs_ref, d2_ref):
    q = q_ref[...]
    pts = pts_ref[...]
    ptsT = ptsT_ref[...]
    q2 = jnp.sum(q * q, axis=1, keepdims=True)
    p2 = jnp.sum(pts * pts, axis=1, keepdims=True).reshape(1, _K)
    qp = jax.lax.dot_general(
        q, ptsT, (((1,), (0,)), ((), ())),
        preferred_element_type=jnp.float32,
        precision=jax.lax.Precision.DEFAULT,
    )
    d2_ref[...] = q2 + p2 - 2.0 * qp

    iota = jax.lax.broadcasted_iota(jnp.int32, (_Q_BLK, _K), 1)
    for i in range(_K_NB):
        d2 = d2_ref[...]
        m = jnp.min(d2, axis=1, keepdims=True)
        idx = jnp.min(jnp.where(d2 == m, iota, _K), axis=1, keepdims=True)
        sel = iota == idx
        d2_ref[...] = jnp.where(sel, jnp.inf, d2)
        valid = m <= _RADIUS2
        map_ref[:, i] = jnp.where(valid, idx, -1)[:, 0]
        oh = jnp.where(sel & valid, 1.0, 0.0)
        outs_ref[:, i, :] = jax.lax.dot_general(
            oh, pts, (((1,), (0,)), ((), ())),
            preferred_element_type=jnp.float32,
            precision=jax.lax.Precision.HIGHEST,
        )


@functools.partial(jax.jit, static_argnames=())
def kernel(x, p_grid):
    q = x[0]
    n = q.shape[0]
    pts = p_grid.reshape(_K, 3)
    ptsT = pts.T
    n_pad = ((n + _Q_BLK - 1) // _Q_BLK) * _Q_BLK
    if n_pad != n:
        q = jnp.concatenate([q, jnp.zeros((n_pad - n, 3), q.dtype)], axis=0)
    grid = n_pad // _Q_BLK

    map_t, outs_t = pl.pallas_call(
        _bq_body,
        grid=(grid,),
        in_specs=[
            pl.BlockSpec((_Q_BLK, 3), lambda i: (i, 0)),
            pl.BlockSpec((_K, 3), lambda i: (0, 0)),
            pl.BlockSpec((3, _K), lambda i: (0, 0)),
        ],
        out_specs=[
            pl.BlockSpec((_Q_BLK, _K_NB), lambda i: (i, 0)),
            pl.BlockSpec((_Q_BLK, _K_NB, 3), lambda i: (i, 0, 0)),
        ],
        out_shape=[
            jax.ShapeDtypeStruct((n_pad, _K_NB), jnp.int32),
            jax.ShapeDtypeStruct((n_pad, _K_NB, 3), jnp.float32),
        ],
        scratch_shapes=[pltpu.VMEM((_Q_BLK, _K), jnp.float32)],
    )(q, pts, ptsT)

    return (map_t[:n][None], outs_t[:n][None])


# speedup vs baseline: 7.3827x; 4.1731x over previous
import functools

import jax
import jax.numpy as jnp
from jax import lax
from jax.experimental import pallas as pl
from jax.experimental.pallas import tpu as pltpu
from jax.experimental.pallas import tpu_sc as plsc

_RADIUS2 = 0.0625
_K_NB = 10
_K = 8192
_Q_BLK = 400

_NW = 32
_B_PAD = 102400
_ROW = 4


def _bq_body(q_ref, pts_ref, ptsT_ref, map_ref, gidx_ref, d2_ref):
    q = q_ref[...]
    pts = pts_ref[...]
    ptsT = ptsT_ref[...]
    q2 = jnp.sum(q * q, axis=1, keepdims=True)
    p2 = jnp.sum(pts * pts, axis=1, keepdims=True).reshape(1, _K)
    qp = jax.lax.dot_general(
        q, ptsT, (((1,), (0,)), ((), ())),
        preferred_element_type=jnp.float32,
        precision=jax.lax.Precision.DEFAULT,
    )
    d2_ref[...] = q2 + p2 - 2.0 * qp

    iota = jax.lax.broadcasted_iota(jnp.int32, (_Q_BLK, _K), 1)
    for i in range(_K_NB):
        d2 = d2_ref[...]
        m = jnp.min(d2, axis=1, keepdims=True)
        idx = jnp.min(jnp.where(d2 == m, iota, _K), axis=1, keepdims=True)
        d2_ref[...] = jnp.where(iota == idx, jnp.inf, d2)
        valid = m <= _RADIUS2
        map_ref[:, i] = jnp.where(valid, idx, -1)[:, 0]
        gidx_ref[:, i] = jnp.where(valid, idx, _K)[:, 0]


def _sc_gather(table_hbm, gidx_hbm, out_hbm, tab_v, idx_v, rx_v, ry_v, rz_v):
    wid = lax.axis_index("s") * 2 + lax.axis_index("c")
    bpw = _B_PAD // _NW
    base = wid * bpw
    pltpu.sync_copy(table_hbm, tab_v)
    pltpu.sync_copy(gidx_hbm.at[pl.ds(base, bpw)], idx_v)

    def body(t, carry):
        o = t * 16
        a = idx_v[pl.ds(o, 16)] * _ROW
        rx_v[pl.ds(o, 16)] = plsc.load_gather(tab_v, [a])
        ry_v[pl.ds(o, 16)] = plsc.load_gather(tab_v, [a + 1])
        rz_v[pl.ds(o, 16)] = plsc.load_gather(tab_v, [a + 2])
        return carry

    lax.fori_loop(0, bpw // 16, body, 0)
    pltpu.sync_copy(rx_v, out_hbm.at[pl.ds(base, bpw)])
    pltpu.sync_copy(ry_v, out_hbm.at[pl.ds(_B_PAD + base, bpw)])
    pltpu.sync_copy(rz_v, out_hbm.at[pl.ds(2 * _B_PAD + base, bpw)])


@functools.partial(jax.jit, static_argnames=())
def kernel(x, p_grid):
    q = x[0]
    n = q.shape[0]
    pts = p_grid.reshape(_K, 3)
    ptsT = pts.T
    n_pad = ((n + _Q_BLK - 1) // _Q_BLK) * _Q_BLK
    if n_pad != n:
        q = jnp.concatenate([q, jnp.zeros((n_pad - n, 3), q.dtype)], axis=0)
    grid = n_pad // _Q_BLK

    map_t, gidx = pl.pallas_call(
        _bq_body,
        grid=(grid,),
        in_specs=[
            pl.BlockSpec((_Q_BLK, 3), lambda i: (i, 0)),
            pl.BlockSpec((_K, 3), lambda i: (0, 0)),
            pl.BlockSpec((3, _K), lambda i: (0, 0)),
        ],
        out_specs=[
            pl.BlockSpec((_Q_BLK, _K_NB), lambda i: (i, 0)),
            pl.BlockSpec((_Q_BLK, _K_NB), lambda i: (i, 0)),
        ],
        out_shape=[
            jax.ShapeDtypeStruct((n_pad, _K_NB), jnp.int32),
            jax.ShapeDtypeStruct((n_pad, _K_NB), jnp.int32),
        ],
        scratch_shapes=[pltpu.VMEM((_Q_BLK, _K), jnp.float32)],
    )(q, pts, ptsT)

    mapping = map_t[:n][None]

    table = jnp.zeros((_K + 8, _ROW), jnp.float32).at[:_K, :3].set(pts)
    table_flat = table.reshape((_K + 8) * _ROW)
    gidx_flat = gidx[:n].reshape(n * _K_NB)
    gidx_flat = jnp.concatenate(
        [gidx_flat, jnp.full((_B_PAD - n * _K_NB,), _K, jnp.int32)])

    bpw = _B_PAD // _NW
    mesh = plsc.VectorSubcoreMesh(core_axis_name="c", subcore_axis_name="s")
    gathered = pl.kernel(
        _sc_gather,
        mesh=mesh,
        compiler_params=pltpu.CompilerParams(needs_layout_passes=False),
        out_type=jax.ShapeDtypeStruct((3 * _B_PAD,), jnp.float32),
        scratch_types=[
            pltpu.VMEM(((_K + 8) * _ROW,), jnp.float32),
            pltpu.VMEM((bpw,), jnp.int32),
            pltpu.VMEM((bpw,), jnp.float32),
            pltpu.VMEM((bpw,), jnp.float32),
            pltpu.VMEM((bpw,), jnp.float32),
        ],
    )(table_flat, gidx_flat)

    outs = gathered.reshape(3, _B_PAD).T[:n * _K_NB].reshape(n, _K_NB, 3)[None]
    return (mapping, outs)


# speedup vs baseline: 7.3874x; 1.0006x over previous
import functools

import jax
import jax.numpy as jnp
from jax import lax
from jax.experimental import pallas as pl
from jax.experimental.pallas import tpu as pltpu
from jax.experimental.pallas import tpu_sc as plsc

_RADIUS2 = 0.0625
_K_NB = 10
_K = 8192
_Q_BLK = 400

_NW = 32
_B_PAD = 102400
_ROW = 4


def _bq_body(q_ref, pts_ref, ptsT_ref, map_ref, gidx_ref):
    q = q_ref[...]
    pts = pts_ref[...]
    ptsT = ptsT_ref[...]
    q2 = jnp.sum(q * q, axis=1, keepdims=True)
    p2 = jnp.sum(pts * pts, axis=1, keepdims=True).reshape(1, _K)
    qp = jax.lax.dot_general(
        q, ptsT, (((1,), (0,)), ((), ())),
        preferred_element_type=jnp.float32,
        precision=jax.lax.Precision.DEFAULT,
    )
    work = q2 + p2 - 2.0 * qp

    iota = jax.lax.broadcasted_iota(jnp.int32, (_Q_BLK, _K), 1)
    idx = None
    for i in range(_K_NB):
        if idx is not None:
            work = jnp.where(iota == idx, jnp.inf, work)
        m = jnp.min(work, axis=1, keepdims=True)
        idx = jnp.min(jnp.where(work == m, iota, _K), axis=1, keepdims=True)
        valid = m <= _RADIUS2
        map_ref[:, i] = jnp.where(valid, idx, -1)[:, 0]
        gidx_ref[:, i] = jnp.where(valid, idx, _K)[:, 0]


def _sc_gather(table_hbm, gidx_hbm, out_hbm, tab_v, idx_v, rx_v, ry_v, rz_v):
    wid = lax.axis_index("s") * 2 + lax.axis_index("c")
    bpw = _B_PAD // _NW
    base = wid * bpw
    pltpu.sync_copy(table_hbm, tab_v)
    pltpu.sync_copy(gidx_hbm.at[pl.ds(base, bpw)], idx_v)

    def body(t, carry):
        o = t * 16
        a = idx_v[pl.ds(o, 16)] * _ROW
        rx_v[pl.ds(o, 16)] = plsc.load_gather(tab_v, [a])
        ry_v[pl.ds(o, 16)] = plsc.load_gather(tab_v, [a + 1])
        rz_v[pl.ds(o, 16)] = plsc.load_gather(tab_v, [a + 2])
        return carry

    lax.fori_loop(0, bpw // 16, body, 0)
    pltpu.sync_copy(rx_v, out_hbm.at[pl.ds(base, bpw)])
    pltpu.sync_copy(ry_v, out_hbm.at[pl.ds(_B_PAD + base, bpw)])
    pltpu.sync_copy(rz_v, out_hbm.at[pl.ds(2 * _B_PAD + base, bpw)])


@functools.partial(jax.jit, static_argnames=())
def kernel(x, p_grid):
    q = x[0]
    n = q.shape[0]
    pts = p_grid.reshape(_K, 3)
    ptsT = pts.T
    n_pad = ((n + _Q_BLK - 1) // _Q_BLK) * _Q_BLK
    if n_pad != n:
        q = jnp.concatenate([q, jnp.zeros((n_pad - n, 3), q.dtype)], axis=0)
    grid = n_pad // _Q_BLK

    map_t, gidx = pl.pallas_call(
        _bq_body,
        grid=(grid,),
        in_specs=[
            pl.BlockSpec((_Q_BLK, 3), lambda i: (i, 0)),
            pl.BlockSpec((_K, 3), lambda i: (0, 0)),
            pl.BlockSpec((3, _K), lambda i: (0, 0)),
        ],
        out_specs=[
            pl.BlockSpec((_Q_BLK, _K_NB), lambda i: (i, 0)),
            pl.BlockSpec((_Q_BLK, _K_NB), lambda i: (i, 0)),
        ],
        out_shape=[
            jax.ShapeDtypeStruct((n_pad, _K_NB), jnp.int32),
            jax.ShapeDtypeStruct((n_pad, _K_NB), jnp.int32),
        ],
    )(q, pts, ptsT)

    mapping = map_t[:n][None]

    table = jnp.zeros((_K + 8, _ROW), jnp.float32).at[:_K, :3].set(pts)
    table_flat = table.reshape((_K + 8) * _ROW)
    gidx_flat = gidx[:n].reshape(n * _K_NB)
    gidx_flat = jnp.concatenate(
        [gidx_flat, jnp.full((_B_PAD - n * _K_NB,), _K, jnp.int32)])

    bpw = _B_PAD // _NW
    mesh = plsc.VectorSubcoreMesh(core_axis_name="c", subcore_axis_name="s")
    gathered = pl.kernel(
        _sc_gather,
        mesh=mesh,
        compiler_params=pltpu.CompilerParams(needs_layout_passes=False),
        out_type=jax.ShapeDtypeStruct((3 * _B_PAD,), jnp.float32),
        scratch_types=[
            pltpu.VMEM(((_K + 8) * _ROW,), jnp.float32),
            pltpu.VMEM((bpw,), jnp.int32),
            pltpu.VMEM((bpw,), jnp.float32),
            pltpu.VMEM((bpw,), jnp.float32),
            pltpu.VMEM((bpw,), jnp.float32),
        ],
    )(table_flat, gidx_flat)

    outs = gathered.reshape(3, _B_PAD).T[:n * _K_NB].reshape(n, _K_NB, 3)[None]
    return (mapping, outs)


# speedup vs baseline: 7.8661x; 1.0648x over previous
import functools

import jax
import jax.numpy as jnp
from jax import lax
from jax.experimental import pallas as pl
from jax.experimental.pallas import tpu as pltpu
from jax.experimental.pallas import tpu_sc as plsc

_RADIUS2 = 0.0625
_K_NB = 10
_K = 8192
_Q_BLK = 400

_NW = 32
_B_PAD = 102400
_ROW = 4


def _bq_body(q_ref, pts_ref, ptsT_ref, map_ref, gidx_ref):
    q = q_ref[...]
    pts = pts_ref[...]
    ptsT = ptsT_ref[...]
    q2 = jnp.sum(q * q, axis=1, keepdims=True)
    p2 = jnp.sum(pts * pts, axis=1, keepdims=True).reshape(1, _K)
    qp = jax.lax.dot_general(
        q.astype(jnp.bfloat16), ptsT.astype(jnp.bfloat16),
        (((1,), (0,)), ((), ())),
        preferred_element_type=jnp.float32,
        precision=jax.lax.Precision.DEFAULT,
    )
    d2 = q2 + p2 - 2.0 * qp

    h = _K // 2
    l, r = d2[:, :h], d2[:, h:]
    piota = jax.lax.broadcasted_iota(jnp.int32, (_Q_BLK, h), 1)
    le = l <= r
    minv = jnp.where(le, l, r)
    mini = jnp.where(le, piota, piota + h)
    maxv = jnp.where(le, r, l)
    maxi = jnp.where(le, piota + h, piota)

    pidx = None
    for i in range(_K_NB):
        if pidx is not None:
            hit = piota == pidx
            minv = jnp.where(hit, maxv, minv)
            mini = jnp.where(hit, maxi, mini)
            maxv = jnp.where(hit, jnp.inf, maxv)
        m = jnp.min(minv, axis=1, keepdims=True)
        idx = jnp.min(jnp.where(minv == m, mini, _K), axis=1, keepdims=True)
        pidx = jnp.bitwise_and(idx, h - 1)
        valid = m <= _RADIUS2
        map_ref[:, i] = jnp.where(valid, idx, -1)[:, 0]
        gidx_ref[:, i] = jnp.where(valid, idx, _K)[:, 0]


def _sc_gather(table_hbm, gidx_hbm, out_hbm, tab_v, idx_v, rx_v, ry_v, rz_v):
    wid = lax.axis_index("s") * 2 + lax.axis_index("c")
    bpw = _B_PAD // _NW
    base = wid * bpw
    pltpu.sync_copy(table_hbm, tab_v)
    pltpu.sync_copy(gidx_hbm.at[pl.ds(base, bpw)], idx_v)

    def body(t, carry):
        o = t * 16
        a = idx_v[pl.ds(o, 16)] * _ROW
        rx_v[pl.ds(o, 16)] = plsc.load_gather(tab_v, [a])
        ry_v[pl.ds(o, 16)] = plsc.load_gather(tab_v, [a + 1])
        rz_v[pl.ds(o, 16)] = plsc.load_gather(tab_v, [a + 2])
        return carry

    lax.fori_loop(0, bpw // 16, body, 0)
    pltpu.sync_copy(rx_v, out_hbm.at[pl.ds(base, bpw)])
    pltpu.sync_copy(ry_v, out_hbm.at[pl.ds(_B_PAD + base, bpw)])
    pltpu.sync_copy(rz_v, out_hbm.at[pl.ds(2 * _B_PAD + base, bpw)])


@functools.partial(jax.jit, static_argnames=())
def kernel(x, p_grid):
    q = x[0]
    n = q.shape[0]
    pts = p_grid.reshape(_K, 3)
    ptsT = pts.T
    n_pad = ((n + _Q_BLK - 1) // _Q_BLK) * _Q_BLK
    if n_pad != n:
        q = jnp.concatenate([q, jnp.zeros((n_pad - n, 3), q.dtype)], axis=0)
    grid = n_pad // _Q_BLK

    map_t, gidx = pl.pallas_call(
        _bq_body,
        grid=(grid,),
        in_specs=[
            pl.BlockSpec((_Q_BLK, 3), lambda i: (i, 0)),
            pl.BlockSpec((_K, 3), lambda i: (0, 0)),
            pl.BlockSpec((3, _K), lambda i: (0, 0)),
        ],
        out_specs=[
            pl.BlockSpec((_Q_BLK, _K_NB), lambda i: (i, 0)),
            pl.BlockSpec((_Q_BLK, _K_NB), lambda i: (i, 0)),
        ],
        out_shape=[
            jax.ShapeDtypeStruct((n_pad, _K_NB), jnp.int32),
            jax.ShapeDtypeStruct((n_pad, _K_NB), jnp.int32),
        ],
    )(q, pts, ptsT)

    mapping = map_t[:n][None]

    table = jnp.zeros((_K + 8, _ROW), jnp.float32).at[:_K, :3].set(pts)
    table_flat = table.reshape((_K + 8) * _ROW)
    gidx_flat = gidx[:n].reshape(n * _K_NB)
    gidx_flat = jnp.concatenate(
        [gidx_flat, jnp.full((_B_PAD - n * _K_NB,), _K, jnp.int32)])

    bpw = _B_PAD // _NW
    mesh = plsc.VectorSubcoreMesh(core_axis_name="c", subcore_axis_name="s")
    gathered = pl.kernel(
        _sc_gather,
        mesh=mesh,
        compiler_params=pltpu.CompilerParams(needs_layout_passes=False),
        out_type=jax.ShapeDtypeStruct((3 * _B_PAD,), jnp.float32),
        scratch_types=[
            pltpu.VMEM(((_K + 8) * _ROW,), jnp.float32),
            pltpu.VMEM((bpw,), jnp.int32),
            pltpu.VMEM((bpw,), jnp.float32),
            pltpu.VMEM((bpw,), jnp.float32),
            pltpu.VMEM((bpw,), jnp.float32),
        ],
    )(table_flat, gidx_flat)

    outs = gathered.reshape(3, _B_PAD).T[:n * _K_NB].reshape(n, _K_NB, 3)[None]
    return (mapping, outs)
